# baseline (speedup 1.0000x reference)
import jax
import jax.numpy as jnp
from jax.experimental import pallas as pl
from jax.experimental.pallas import tpu as pltpu

_TILE_L1 = 512


def _pool_body(x2_ref, w2_ref, b1_ref, b2_ref, c_ref, *, inv_l2):
    s = jnp.sum(x2_ref[0], axis=0, keepdims=True) * inv_l2
    c = jax.lax.dot_general(s, w2_ref[...], (((1,), (1,)), ((), ())),
                            preferred_element_type=jnp.float32)
    c_ref[0] = c + b1_ref[...] + b2_ref[...]


def _mm_body(x1_ref, w1_ref, c_ref, o_ref):
    x = x1_ref[0].astype(jnp.bfloat16)
    y = jax.lax.dot_general(x, w1_ref[...], (((1,), (1,)), ((), ())),
                            preferred_element_type=jnp.float32)
    o_ref[0] = y + c_ref[0]


def kernel(x1, x2, W1, b1, W2, b2):
    b, l1, d1 = x1.shape
    l2, d2 = x2.shape[1], x2.shape[2]
    d3 = W1.shape[0]
    tile = _TILE_L1
    tiles_per_batch = l1 // tile
    n_tiles = b * tiles_per_batch

    import functools
    c = pl.pallas_call(
        functools.partial(_pool_body, inv_l2=1.0 / l2),
        out_shape=jax.ShapeDtypeStruct((b, 1, d3), jnp.float32),
        grid=(b,),
        in_specs=[
            pl.BlockSpec((1, l2, d2), lambda i: (i, 0, 0)),
            pl.BlockSpec((d3, d2), lambda i: (0, 0)),
            pl.BlockSpec((1, d3), lambda i: (0, 0)),
            pl.BlockSpec((1, d3), lambda i: (0, 0)),
        ],
        out_specs=pl.BlockSpec((1, 1, d3), lambda i: (i, 0, 0)),
        compiler_params=pltpu.CompilerParams(
            dimension_semantics=("parallel",),
            vmem_limit_bytes=56 * 1024 * 1024,
        ),
        name="x2_mean_pool",
    )(x2, W2, b1.reshape(1, d3), b2.reshape(1, d3))

    x1_t = x1.reshape(n_tiles, tile, d1)
    w1_bf = W1.astype(jnp.bfloat16)
    out = pl.pallas_call(
        _mm_body,
        out_shape=jax.ShapeDtypeStruct((n_tiles, tile, d3), jnp.float32),
        grid=(n_tiles,),
        in_specs=[
            pl.BlockSpec((1, tile, d1), lambda t: (t, 0, 0)),
            pl.BlockSpec((d3, d1), lambda t: (0, 0)),
            pl.BlockSpec((1, 1, d3), lambda t: (t // tiles_per_batch, 0, 0)),
        ],
        out_specs=pl.BlockSpec((1, tile, d3), lambda t: (t, 0, 0)),
        compiler_params=pltpu.CompilerParams(
            dimension_semantics=("parallel",),
            vmem_limit_bytes=56 * 1024 * 1024,
        ),
        name="proj_add",
    )(x1_t, w1_bf, c)
    return out.reshape(b, l1, d3)


# speedup vs baseline: 1.1828x; 1.1828x over previous
import jax
import jax.numpy as jnp
from jax.experimental import pallas as pl
from jax.experimental.pallas import tpu as pltpu

_TILE_L1 = 1024


def _pool_body(x2_ref, w2_ref, b1_ref, b2_ref, c_ref, *, inv_l2):
    s = jnp.sum(x2_ref[0], axis=0, keepdims=True) * inv_l2
    c = jax.lax.dot_general(s, w2_ref[...], (((1,), (1,)), ((), ())),
                            preferred_element_type=jnp.float32)
    c_ref[0] = c + b1_ref[...] + b2_ref[...]


def _mm_body(x1_ref, w1_ref, c_ref, o_ref):
    x = x1_ref[0].astype(jnp.bfloat16)
    y = jax.lax.dot_general(x, w1_ref[...], (((1,), (1,)), ((), ())),
                            preferred_element_type=jnp.float32)
    o_ref[0] = y + c_ref[0]


def kernel(x1, x2, W1, b1, W2, b2):
    b, l1, d1 = x1.shape
    l2, d2 = x2.shape[1], x2.shape[2]
    d3 = W1.shape[0]
    tile = _TILE_L1
    tiles_per_batch = l1 // tile
    n_tiles = b * tiles_per_batch

    import functools
    c = pl.pallas_call(
        functools.partial(_pool_body, inv_l2=1.0 / l2),
        out_shape=jax.ShapeDtypeStruct((b, 1, d3), jnp.float32),
        grid=(b,),
        in_specs=[
            pl.BlockSpec((1, l2, d2), lambda i: (i, 0, 0)),
            pl.BlockSpec((d3, d2), lambda i: (0, 0)),
            pl.BlockSpec((1, d3), lambda i: (0, 0)),
            pl.BlockSpec((1, d3), lambda i: (0, 0)),
        ],
        out_specs=pl.BlockSpec((1, 1, d3), lambda i: (i, 0, 0)),
        compiler_params=pltpu.CompilerParams(
            dimension_semantics=("parallel",),
            vmem_limit_bytes=56 * 1024 * 1024,
        ),
        name="x2_mean_pool",
    )(x2, W2, b1.reshape(1, d3), b2.reshape(1, d3))

    x1_t = x1.reshape(n_tiles, tile, d1)
    w1_bf = W1.astype(jnp.bfloat16)
    out = pl.pallas_call(
        _mm_body,
        out_shape=jax.ShapeDtypeStruct((n_tiles, tile, d3), jnp.float32),
        grid=(n_tiles,),
        in_specs=[
            pl.BlockSpec((1, tile, d1), lambda t: (t, 0, 0)),
            pl.BlockSpec((d3, d1), lambda t: (0, 0)),
            pl.BlockSpec((1, 1, d3), lambda t: (t // tiles_per_batch, 0, 0)),
        ],
        out_specs=pl.BlockSpec((1, tile, d3), lambda t: (t, 0, 0)),
        compiler_params=pltpu.CompilerParams(
            dimension_semantics=("parallel",),
            vmem_limit_bytes=56 * 1024 * 1024,
        ),
        name="proj_add",
    )(x1_t, w1_bf, c)
    return out.reshape(b, l1, d3)


# speedup vs baseline: 1.2956x; 1.0953x over previous
import jax
import jax.numpy as jnp
from jax.experimental import pallas as pl
from jax.experimental.pallas import tpu as pltpu

_TILE_L1 = 2048


def _pool_body(x2_ref, w2_ref, b1_ref, b2_ref, c_ref, *, inv_l2):
    s = jnp.sum(x2_ref[0], axis=0, keepdims=True) * inv_l2
    c = jax.lax.dot_general(s, w2_ref[...], (((1,), (1,)), ((), ())),
                            preferred_element_type=jnp.float32)
    c_ref[0] = c + b1_ref[...] + b2_ref[...]


def _mm_body(x1_ref, w1_ref, c_ref, o_ref):
    x = x1_ref[0].astype(jnp.bfloat16)
    w = w1_ref[...].astype(jnp.bfloat16)
    y = jax.lax.dot_general(x, w, (((1,), (1,)), ((), ())),
                            preferred_element_type=jnp.float32)
    o_ref[0] = y + c_ref[0]


def kernel(x1, x2, W1, b1, W2, b2):
    b, l1, d1 = x1.shape
    l2, d2 = x2.shape[1], x2.shape[2]
    d3 = W1.shape[0]
    tile = _TILE_L1
    tiles_per_batch = l1 // tile
    n_tiles = b * tiles_per_batch

    import functools
    c = pl.pallas_call(
        functools.partial(_pool_body, inv_l2=1.0 / l2),
        out_shape=jax.ShapeDtypeStruct((b, 1, d3), jnp.float32),
        grid=(b,),
        in_specs=[
            pl.BlockSpec((1, l2, d2), lambda i: (i, 0, 0)),
            pl.BlockSpec((d3, d2), lambda i: (0, 0)),
            pl.BlockSpec((1, d3), lambda i: (0, 0)),
            pl.BlockSpec((1, d3), lambda i: (0, 0)),
        ],
        out_specs=pl.BlockSpec((1, 1, d3), lambda i: (i, 0, 0)),
        compiler_params=pltpu.CompilerParams(
            dimension_semantics=("parallel",),
            vmem_limit_bytes=56 * 1024 * 1024,
        ),
        name="x2_mean_pool",
    )(x2, W2, b1.reshape(1, d3), b2.reshape(1, d3))

    x1_t = x1.reshape(n_tiles, tile, d1)
    out = pl.pallas_call(
        _mm_body,
        out_shape=jax.ShapeDtypeStruct((n_tiles, tile, d3), jnp.float32),
        grid=(n_tiles,),
        in_specs=[
            pl.BlockSpec((1, tile, d1), lambda t: (t, 0, 0)),
            pl.BlockSpec((d3, d1), lambda t: (0, 0)),
            pl.BlockSpec((1, 1, d3), lambda t: (t // tiles_per_batch, 0, 0)),
        ],
        out_specs=pl.BlockSpec((1, tile, d3), lambda t: (t, 0, 0)),
        compiler_params=pltpu.CompilerParams(
            dimension_semantics=("parallel",),
            vmem_limit_bytes=56 * 1024 * 1024,
        ),
        name="proj_add",
    )(x1_t, W1, c)
    return out.reshape(b, l1, d3)
